# baseline (speedup 1.0000x reference)
import functools

import jax
import jax.numpy as jnp
from jax import lax
from jax.experimental import pallas as pl
from jax.experimental.pallas import tpu as pltpu
from jax.experimental.pallas import tpu_sc as plsc

B = 16384
D = 1024
MAXH = 128
MAXW = 128
NC = 2
NS = 16
NW = NC * NS
RPW = B // NW
CH = 32
NCH = RPW // CH
NBUF = 4
LANES = 16

_MESH = plsc.VectorSubcoreMesh(core_axis_name="c", subcore_axis_name="s")


def _ring_copy(table_hbm, out_hbm, base, bufs, gsems, osems,
               idx_for=None, nch=NCH):
    nbuf = len(bufs)
    gathers = [None] * nbuf
    out_pending = [None] * nbuf

    def start_gather(c):
        b = c % nbuf
        if idx_for is None:
            src = table_hbm.at[pl.ds(base + c * CH, CH)]
        else:
            src = table_hbm.at[idx_for(c, b)]
        gathers[b] = pltpu.async_copy(src, bufs[b], gsems[b])

    for c in range(nbuf - 1):
        start_gather(c)
    for c in range(nch):
        b = c % nbuf
        gathers[b].wait()
        out_pending[b] = pltpu.async_copy(
            bufs[b], out_hbm.at[pl.ds(base + c * CH, CH)], osems[b])
        n = c + nbuf - 1
        if n < nch:
            bn = n % nbuf
            if out_pending[bn] is not None:
                out_pending[bn].wait()
                out_pending[bn] = None
            start_gather(n)
    for b in range(nbuf):
        if out_pending[b] is not None:
            out_pending[b].wait()


@functools.partial(
    pl.kernel,
    out_type=jax.ShapeDtypeStruct((B, D), jnp.float32),
    mesh=_MESH,
    scratch_types=(
        [pltpu.VMEM_SHARED((NS, 2, CH, D), jnp.float32)]
        + [pltpu.VMEM((CH, D), jnp.float32) for _ in range(2)]
        + [pltpu.SemaphoreType.DMA for _ in range(2 * NBUF)]
    ),
)
def _pos_copy(table_hbm, out_hbm, shared, tbuf0, tbuf1,
              g0, g1, g2, g3, o0, o1, o2, o3):
    sid = lax.axis_index("s")
    wid = sid * NC + lax.axis_index("c")
    base = wid * RPW
    bufs = (shared.at[sid, 0], tbuf0, shared.at[sid, 1], tbuf1)
    _ring_copy(table_hbm, out_hbm, base, bufs,
               (g0, g1, g2, g3), (o0, o1, o2, o3))


@functools.partial(
    pl.kernel,
    out_type=jax.ShapeDtypeStruct((B, D), jnp.float32),
    mesh=_MESH,
    scratch_types=(
        [pltpu.VMEM((CH,), jnp.int32) for _ in range(3)]
        + [pltpu.VMEM((LANES,), jnp.int32),
           pltpu.VMEM((LANES,), jnp.int32)]
        + [pltpu.VMEM((CH, D), jnp.float32) for _ in range(3)]
        + [pltpu.SemaphoreType.DMA for _ in range(6)]
    ),
)
def _pos_gather(hmax_hbm, wmax_hbm, table_hbm, out_hbm,
                idx0, idx1, idx2, hv_v, wv_v,
                tbuf0, tbuf1, tbuf2,
                g0, g1, g2, o0, o1, o2):
    wid = lax.axis_index("s") * NC + lax.axis_index("c")
    base = wid * RPW

    pltpu.sync_copy(hmax_hbm, hv_v)
    pltpu.sync_copy(wmax_hbm, wv_v)
    hmax = hv_v[...]
    wmax = wv_v[...]

    lane = lax.iota(jnp.int32, LANES)
    idxs = (idx0, idx1, idx2)

    def idx_for(c, b):
        for i in range(CH // LANES):
            p = lane + (base + c * CH + i * LANES)
            row = lax.shift_right_logical(p, 7)
            col = lax.bitwise_and(p, MAXW - 1)
            valid = (row < hmax) & (col < wmax)
            idxs[b][pl.ds(i * LANES, LANES)] = jnp.where(valid, p, 0)
        return idxs[b]

    _ring_copy(table_hbm, out_hbm, base, (tbuf0, tbuf1, tbuf2),
               (g0, g1, g2), (o0, o1, o2), idx_for=idx_for)


def kernel(grid_size, pos_embed_table):
    table = pos_embed_table.reshape(B, D)
    gs = grid_size.astype(jnp.int32)
    full = (gs[0] >= MAXH) & (gs[1] >= MAXW)
    hmax = jnp.broadcast_to(gs[0], (LANES,))
    wmax = jnp.broadcast_to(gs[1], (LANES,))
    out = lax.cond(
        full,
        lambda h, w, t: _pos_copy(t),
        lambda h, w, t: _pos_gather(h, w, t),
        hmax, wmax, table)
    return out.reshape(1, B, D)
